# baseline (speedup 1.0000x reference)
import functools

import jax
import jax.numpy as jnp
from jax import lax
from jax.experimental import pallas as pl
from jax.experimental.pallas import tpu as pltpu
from jax.experimental.pallas import tpu_sc as plsc

BATCH = 16384
SEQ_LEN = 200
VOCAB = 1000000
LANE = 128

_info = plsc.get_sparse_core_info()
NC = _info.num_cores
NS = _info.num_subcores
NW = NC * NS
COLS_PER_W = BATCH // NW
CROWS = 40
NRG = SEQ_LEN // CROWS
NCB = COLS_PER_W // LANE
NCHUNK = NRG * NCB
GROUP = 8
LAG = 2
STG = 12800
SLICE = (VOCAB // NS) // 8 * 8
TAIL = VOCAB - NS * SLICE


def _make_kernel():
    mesh = plsc.VectorSubcoreMesh(core_axis_name="c", subcore_axis_name="s")

    @functools.partial(
        pl.kernel,
        mesh=mesh,
        out_type=jax.ShapeDtypeStruct((SEQ_LEN, BATCH), jnp.int32),
        scratch_types=[
            pltpu.VMEM_SHARED((VOCAB,), jnp.int32),
            pltpu.VMEM((STG,), jnp.int32),
            [pltpu.VMEM((CROWS, LANE), jnp.int32) for _ in range(2)],
            [pltpu.VMEM((CROWS, LANE), jnp.int32) for _ in range(2)],
            [pltpu.SemaphoreType.DMA for _ in range(2)],
            [pltpu.SemaphoreType.DMA for _ in range(2)],
            [pltpu.SemaphoreType.DMA for _ in range(2)],
        ],
    )
    def gather_kernel(tok_hbm, vocab_hbm, out_hbm, table_sh, stg,
                      idx, val, sems_i, sems_o, sems_g):
        sid = lax.axis_index("s")
        wid = sid * NC + lax.axis_index("c")
        col_base = wid * COLS_PER_W

        def chunk_slice(ref, k):
            rg, cb = k % NRG, k // NRG
            return ref.at[pl.ds(rg * CROWS, CROWS),
                          pl.ds(col_base + cb * LANE, LANE)]

        def in_cp(k, b):
            return pltpu.make_async_copy(chunk_slice(tok_hbm, k), idx[b],
                                         sems_i[b])

        def out_cp(k, b):
            return pltpu.make_async_copy(val[b], chunk_slice(out_hbm, k),
                                         sems_o[b])

        in_cp(0, 0).start()

        n_full, last = divmod(SLICE, STG)
        for j in range(n_full + (1 if last else 0)):
            ln = STG if j < n_full else last
            off = sid * SLICE + j * STG
            pltpu.sync_copy(vocab_hbm.at[pl.ds(off, ln)], stg.at[pl.ds(0, ln)])
            pltpu.sync_copy(stg.at[pl.ds(0, ln)], table_sh.at[pl.ds(off, ln)])

        @pl.when(sid == 0)
        def _():
            off = NS * SLICE
            pltpu.sync_copy(vocab_hbm.at[pl.ds(off, TAIL)],
                            stg.at[pl.ds(0, TAIL)])
            pltpu.sync_copy(stg.at[pl.ds(0, TAIL)],
                            table_sh.at[pl.ds(off, TAIL)])

        plsc.subcore_barrier()

        def src_ref(k):
            return table_sh

        def fire_gathers(k, b):
            src = src_ref(k)

            @pl.loop(0, CROWS, step=GROUP)
            def _(g):
                for u in range(GROUP):
                    pltpu.make_async_copy(
                        src.at[idx[b].at[g + u]],
                        val[b].at[g + u], sems_g[b]).start()

                @pl.when(g >= LAG * GROUP)
                def _():
                    for u in range(GROUP):
                        pltpu.make_async_copy(
                            src.at[idx[b].at[g - LAG * GROUP + u]],
                            val[b].at[g - LAG * GROUP + u], sems_g[b]).wait()

        def drain_gathers(k, b):
            src = src_ref(k)

            @pl.loop(CROWS - LAG * GROUP, CROWS, step=GROUP)
            def _(g):
                for u in range(GROUP):
                    pltpu.make_async_copy(
                        src.at[idx[b].at[g + u]],
                        val[b].at[g + u], sems_g[b]).wait()

        in_cp(1, 1).start()
        in_cp(0, 0).wait()
        fire_gathers(0, 0)
        for k in range(1, NCHUNK):
            b = k % 2
            in_cp(k, b).wait()
            if k >= 2:
                out_cp(k - 2, b).wait()
            fire_gathers(k, b)
            drain_gathers(k - 1, 1 - b)
            if k + 1 < NCHUNK:
                in_cp(k + 1, 1 - b).start()
            out_cp(k - 1, 1 - b).start()
        drain_gathers(NCHUNK - 1, (NCHUNK - 1) % 2)
        out_cp(NCHUNK - 1, (NCHUNK - 1) % 2).start()
        out_cp(NCHUNK - 2, NCHUNK % 2).wait()
        out_cp(NCHUNK - 1, (NCHUNK - 1) % 2).wait()

    return gather_kernel


_gather = _make_kernel()


def kernel(tokens, vocab_map):
    return _gather(tokens.T, vocab_map).T
